# Initial kernel scaffold; baseline (speedup 1.0000x reference)
import jax
import jax.numpy as jnp
from jax.experimental import pallas as pl


def kernel(x, params):
    raise NotImplementedError("write your pallas kernel here")


# speedup vs baseline: 1214.3256x; 1214.3256x over previous
import math

import jax
import jax.numpy as jnp
from jax import lax
from jax.experimental import pallas as pl
from jax.experimental.pallas import tpu as pltpu

N = 524288
D = 16
K = 8
RANGE_MIN = -3.0
RANGE_MAX = 3.0
MIN_BIN_SIZE = 1e-4
MIN_SLOPE = 1e-4
LANES = 128
SPB = LANES // D

ROWS = N // SPB
BLOCK_ROWS = 512


def _rqs_block(pt_ref, x_ref, y_ref, ld_ref):
    pt = pt_ref[...]
    wu = pt[0:K, :]
    hu = pt[K:2 * K, :]
    su = pt[2 * K:2 * K + K + 1, :]

    total = RANGE_MAX - RANGE_MIN
    widths = jax.nn.softmax(wu, axis=0) * (total - K * MIN_BIN_SIZE) + MIN_BIN_SIZE
    heights = jax.nn.softmax(hu, axis=0) * (total - K * MIN_BIN_SIZE) + MIN_BIN_SIZE
    offset = math.log(math.exp(1.0 - MIN_SLOPE) - 1.0)
    slopes = jax.nn.softplus(su + offset) + MIN_SLOPE

    xp = [jnp.full((1, LANES), RANGE_MIN, jnp.float32)]
    yp = [jnp.full((1, LANES), RANGE_MIN, jnp.float32)]
    for j in range(K):
        xp.append(xp[-1] + widths[j:j + 1, :])
        yp.append(yp[-1] + heights[j:j + 1, :])
    sl = [slopes[j:j + 1, :] for j in range(K + 1)]

    inv_w = [1.0 / (widths[k:k + 1, :] + 1e-8) for k in range(K)]
    h_tab = [heights[k:k + 1, :] for k in range(K)]
    s_tab = [h_tab[k] * inv_w[k] for k in range(K)]

    x = x_ref[...]

    m = [x >= xp[j] for j in range(1, K)]

    def gather(tab):
        v = jnp.broadcast_to(tab[0], x.shape)
        for j in range(1, K):
            v = jnp.where(m[j - 1], jnp.broadcast_to(tab[j], x.shape), v)
        return v

    x_k = gather(xp[:K])
    y_k = gather(yp[:K])
    s_k = gather(sl[:K])
    s_k1 = gather(sl[1:K + 1])
    iw = gather(inv_w)
    h = gather(h_tab)
    s = gather(s_tab)

    xi = jnp.clip((x - x_k) * iw, 0.0, 1.0)
    omx = 1.0 - xi
    u = xi * omx
    t = xi * xi
    num = s * t + s_k * u
    den0 = s + (s_k1 + s_k - 2.0 * s) * u
    den = jnp.maximum(jnp.abs(den0), 1e-8) * jnp.sign(den0)
    r = 1.0 / den
    y_sp = y_k + h * (num * r)
    dnum = (s * s) * (s_k1 * t + 2.0 * s * u + s_k * (omx * omx))
    deriv = jnp.maximum(dnum * r * r, 1e-8)

    below = x < RANGE_MIN
    above = x > RANGE_MAX
    sl0 = jnp.broadcast_to(sl[0], x.shape)
    sl8 = jnp.broadcast_to(sl[K], x.shape)
    y_lin_l = (x - RANGE_MIN) * sl0 + RANGE_MIN
    y_lin_r = (x - RANGE_MAX) * sl8 + RANGE_MAX
    y = jnp.where(below, y_lin_l, jnp.where(above, y_lin_r, y_sp))
    dsel = jnp.where(below, sl0, jnp.where(above, sl8, deriv))
    ld = jnp.log(dsel)

    y_ref[...] = y

    red = (lax.broadcasted_iota(jnp.int32, (LANES, SPB), 0) // D
           == lax.broadcasted_iota(jnp.int32, (LANES, SPB), 1)).astype(jnp.float32)
    ld_ref[...] = jax.lax.dot(ld, red, precision=jax.lax.Precision.HIGHEST)


def kernel(x, params):
    xr = x.reshape(ROWS, LANES)
    pt = jnp.tile(params.T, (1, SPB))
    grid = (ROWS // BLOCK_ROWS,)
    y2, ld2 = pl.pallas_call(
        _rqs_block,
        grid=grid,
        in_specs=[
            pl.BlockSpec((3 * K + 1, LANES), lambda i: (0, 0)),
            pl.BlockSpec((BLOCK_ROWS, LANES), lambda i: (i, 0)),
        ],
        out_specs=[
            pl.BlockSpec((BLOCK_ROWS, LANES), lambda i: (i, 0)),
            pl.BlockSpec((BLOCK_ROWS, SPB), lambda i: (i, 0)),
        ],
        out_shape=[
            jax.ShapeDtypeStruct((ROWS, LANES), jnp.float32),
            jax.ShapeDtypeStruct((ROWS, SPB), jnp.float32),
        ],
    )(pt, xr)
    return y2.reshape(N, D), ld2.reshape(N)


# speedup vs baseline: 1259.6841x; 1.0374x over previous
import math

import jax
import jax.numpy as jnp
from jax import lax
from jax.experimental import pallas as pl
from jax.experimental.pallas import tpu as pltpu

N = 524288
D = 16
K = 8
RANGE_MIN = -3.0
RANGE_MAX = 3.0
MIN_BIN_SIZE = 1e-4
MIN_SLOPE = 1e-4
LANES = 128
SPB = LANES // D

ROWS = N // SPB
BLOCK_ROWS = 1024


def _outer(x_hbm, pt_ref, y_hbm, ld_hbm):
    pt = pt_ref[...]
    wu = pt[0:K, :]
    hu = pt[K:2 * K, :]
    su = pt[2 * K:3 * K + 1, :]

    total = RANGE_MAX - RANGE_MIN
    widths = jax.nn.softmax(wu, axis=0) * (total - K * MIN_BIN_SIZE) + MIN_BIN_SIZE
    heights = jax.nn.softmax(hu, axis=0) * (total - K * MIN_BIN_SIZE) + MIN_BIN_SIZE
    offset = math.log(math.exp(1.0 - MIN_SLOPE) - 1.0)
    slopes = jax.nn.softplus(su + offset) + MIN_SLOPE

    xp = [jnp.full((1, LANES), RANGE_MIN, jnp.float32)]
    yp = [jnp.full((1, LANES), RANGE_MIN, jnp.float32)]
    for j in range(K):
        xp.append(xp[-1] + widths[j:j + 1, :])
        yp.append(yp[-1] + heights[j:j + 1, :])
    sl = [slopes[j:j + 1, :] for j in range(K + 1)]

    inv_w = [1.0 / (widths[k:k + 1, :] + 1e-8) for k in range(K)]
    h_tab = [heights[k:k + 1, :] for k in range(K)]

    red = (lax.broadcasted_iota(jnp.int32, (LANES, SPB), 0) // D
           == lax.broadcasted_iota(jnp.int32, (LANES, SPB), 1)).astype(jnp.float32)

    def inner(x_ref, y_ref, ld_ref):
        x = x_ref[...].reshape(BLOCK_ROWS, LANES)
        m = [x >= xp[j] for j in range(1, K)]

        def gather(tab):
            v = jnp.broadcast_to(tab[0], x.shape)
            for j in range(1, K):
                v = jnp.where(m[j - 1], jnp.broadcast_to(tab[j], x.shape), v)
            return v

        x_k = gather(xp[:K])
        y_k = gather(yp[:K])
        s_k = gather(sl[:K])
        s_k1 = gather(sl[1:K + 1])
        iw = gather(inv_w)
        h = gather(h_tab)
        s = h * iw

        xi = jnp.clip((x - x_k) * iw, 0.0, 1.0)
        omx = 1.0 - xi
        u = xi * omx
        t = xi * xi
        num = s * t + s_k * u
        den0 = s + (s_k1 + s_k - 2.0 * s) * u
        den = jnp.maximum(jnp.abs(den0), 1e-8) * jnp.sign(den0)
        r = 1.0 / den
        y_sp = y_k + h * (num * r)
        dnum = (s * s) * (s_k1 * t + 2.0 * s * u + s_k * (omx * omx))
        deriv = jnp.maximum(dnum * r * r, 1e-8)

        below = x < RANGE_MIN
        above = x > RANGE_MAX
        sl0 = jnp.broadcast_to(sl[0], x.shape)
        sl8 = jnp.broadcast_to(sl[K], x.shape)
        y_lin_l = (x - RANGE_MIN) * sl0 + RANGE_MIN
        y_lin_r = (x - RANGE_MAX) * sl8 + RANGE_MAX
        y_ref[...] = jnp.where(below, y_lin_l,
                               jnp.where(above, y_lin_r, y_sp)).reshape(BLOCK_ROWS * LANES)
        dsel = jnp.where(below, sl0, jnp.where(above, sl8, deriv))
        ld = jnp.log(dsel)
        ld_ref[...] = jax.lax.dot(ld, red, precision=jax.lax.Precision.HIGHEST)

    pltpu.emit_pipeline(
        inner,
        grid=(ROWS // BLOCK_ROWS,),
        in_specs=[pl.BlockSpec((BLOCK_ROWS * LANES,), lambda i: (i,))],
        out_specs=[
            pl.BlockSpec((BLOCK_ROWS * LANES,), lambda i: (i,)),
            pl.BlockSpec((BLOCK_ROWS, SPB), lambda i: (i, 0)),
        ],
    )(x_hbm, y_hbm, ld_hbm)


def kernel(x, params):
    pt = jnp.tile(params.T, (1, SPB))
    y, ld = pl.pallas_call(
        _outer,
        in_specs=[
            pl.BlockSpec(memory_space=pltpu.HBM),
            pl.BlockSpec(memory_space=pltpu.VMEM),
        ],
        out_specs=[
            pl.BlockSpec(memory_space=pltpu.HBM),
            pl.BlockSpec(memory_space=pltpu.HBM),
        ],
        out_shape=[
            jax.ShapeDtypeStruct((N * D,), jnp.float32),
            jax.ShapeDtypeStruct((ROWS, SPB), jnp.float32),
        ],
    )(x.reshape(N * D), pt)
    return y.reshape(N, D), ld.reshape(N)


# speedup vs baseline: 2855.4425x; 2.2668x over previous
import math

import jax
import jax.numpy as jnp
from jax.experimental import pallas as pl
from jax.experimental.pallas import tpu as pltpu

N = 524288
D = 16
K = 8
RANGE_MIN = -3.0
RANGE_MAX = 3.0
MIN_BIN_SIZE = 1e-4
MIN_SLOPE = 1e-4

BL = 16384


def _rqs_block(p_ref, x_ref, y_ref, ld_ref):
    p = p_ref[...]
    wu = p[:, 0:K]
    hu = p[:, K:2 * K]
    su = p[:, 2 * K:3 * K + 1]

    total = RANGE_MAX - RANGE_MIN
    widths = jax.nn.softmax(wu, axis=-1) * (total - K * MIN_BIN_SIZE) + MIN_BIN_SIZE
    heights = jax.nn.softmax(hu, axis=-1) * (total - K * MIN_BIN_SIZE) + MIN_BIN_SIZE
    offset = math.log(math.exp(1.0 - MIN_SLOPE) - 1.0)
    slopes = jax.nn.softplus(su + offset) + MIN_SLOPE

    xp = [jnp.full((D, 1), RANGE_MIN, jnp.float32)]
    yp = [jnp.full((D, 1), RANGE_MIN, jnp.float32)]
    for j in range(K):
        xp.append(xp[-1] + widths[:, j:j + 1])
        yp.append(yp[-1] + heights[:, j:j + 1])
    sl = [slopes[:, j:j + 1] for j in range(K + 1)]

    inv_w = [1.0 / (widths[:, k:k + 1] + 1e-8) for k in range(K)]
    h_tab = [heights[:, k:k + 1] for k in range(K)]

    x = x_ref[...]
    m = [x >= xp[j] for j in range(1, K)]

    def gather(tab):
        v = jnp.broadcast_to(tab[0], x.shape)
        for j in range(1, K):
            v = jnp.where(m[j - 1], jnp.broadcast_to(tab[j], x.shape), v)
        return v

    x_k = gather(xp[:K])
    y_k = gather(yp[:K])
    s_k = gather(sl[:K])
    s_k1 = gather(sl[1:K + 1])
    iw = gather(inv_w)
    h = gather(h_tab)
    s = h * iw

    xi = jnp.clip((x - x_k) * iw, 0.0, 1.0)
    omx = 1.0 - xi
    u = xi * omx
    t = xi * xi
    num = s * t + s_k * u
    den0 = s + (s_k1 + s_k - 2.0 * s) * u
    den = jnp.maximum(jnp.abs(den0), 1e-8) * jnp.sign(den0)
    r = 1.0 / den
    y_sp = y_k + h * (num * r)
    dnum = (s * s) * (s_k1 * t + 2.0 * s * u + s_k * (omx * omx))
    deriv = jnp.maximum(dnum * r * r, 1e-8)

    below = x < RANGE_MIN
    above = x > RANGE_MAX
    sl0 = jnp.broadcast_to(sl[0], x.shape)
    sl8 = jnp.broadcast_to(sl[K], x.shape)
    y_lin_l = (x - RANGE_MIN) * sl0 + RANGE_MIN
    y_lin_r = (x - RANGE_MAX) * sl8 + RANGE_MAX
    y_ref[...] = jnp.where(below, y_lin_l, jnp.where(above, y_lin_r, y_sp))
    dsel = jnp.where(below, sl0, jnp.where(above, sl8, deriv))
    ld_ref[...] = jnp.sum(jnp.log(dsel), axis=0)


def kernel(x, params):
    xt = x.T
    yt, ld = pl.pallas_call(
        _rqs_block,
        grid=(N // BL,),
        in_specs=[
            pl.BlockSpec((D, 3 * K + 1), lambda i: (0, 0)),
            pl.BlockSpec((D, BL), lambda i: (0, i)),
        ],
        out_specs=[
            pl.BlockSpec((D, BL), lambda i: (0, i)),
            pl.BlockSpec((BL,), lambda i: (i,)),
        ],
        out_shape=[
            jax.ShapeDtypeStruct((D, N), jnp.float32),
            jax.ShapeDtypeStruct((N,), jnp.float32),
        ],
        compiler_params=pltpu.CompilerParams(
            dimension_semantics=("parallel",),
        ),
    )(params, xt)
    return yt.T, ld
